# baseline (speedup 1.0000x reference)
import functools

import jax
import jax.numpy as jnp
from jax.experimental import pallas as pl

BIG = 3.0e38
PADV = 1.0e18
B = 8
LANES = 128
G = 16


def _batcher_pairs(n):
    pairs = []
    p = 1
    while p < n:
        k = p
        while k >= 1:
            for j in range(k % p, n - k, 2 * k):
                for i in range(0, min(k, n - j - k)):
                    if (i + j) // (2 * p) == (i + j + k) // (2 * p):
                        pairs.append((i + j, i + j + k))
            k //= 2
        p *= 2
    return pairs


_SORT16 = _batcher_pairs(16)
_BITONIC16 = [(j, j + d) for d in (8, 4, 2, 1) for j in range(16) if (j & d) == 0]


def _knn_kernel(kx_ref, ky_ref, qx_ref, qy_ref, o_ref, *, n_groups, n_sub):
    i = pl.program_id(0)
    lane_iota = jax.lax.broadcasted_iota(jnp.int32, (1, LANES), 1)
    sub_iota = jax.lax.broadcasted_iota(jnp.int32, (B, 1), 0)
    for t in range(n_sub):
        _knn_block(kx_ref, ky_ref, qx_ref[t], qy_ref[t], o_ref, t,
                   (i * n_sub + t) * B + sub_iota, lane_iota, n_groups)


def _knn_block(kx_ref, ky_ref, qx, qy, o_ref, t, qidx, lane_iota, n_groups):

    def dist_row(s):
        kxr = kx_ref[s : s + 1, :]
        kyr = ky_ref[s : s + 1, :]
        dx = qx - kxr
        dy = qy - kyr
        d2 = dx * dx + dy * dy
        kidx = s * LANES + lane_iota
        return jnp.where(kidx == qidx, BIG, d2)

    carry = [jnp.full((B, LANES), BIG, jnp.float32) for _ in range(G)]
    for g in range(n_groups):
        rows = [dist_row(g * G + t) for t in range(G)]
        for a, b in _SORT16:
            lo = jnp.minimum(rows[a], rows[b])
            hi = jnp.maximum(rows[a], rows[b])
            rows[a], rows[b] = lo, hi
        lows = [jnp.minimum(carry[j], rows[G - 1 - j]) for j in range(G)]
        for a, b in _BITONIC16:
            lo = jnp.minimum(lows[a], lows[b])
            hi = jnp.maximum(lows[a], lows[b])
            lows[a], lows[b] = lo, hi
        carry = lows

    for d in (1, 2, 4, 8, 16, 32, 64):
        rolled = [jnp.roll(c, -d, axis=1) for c in carry]
        lows = [jnp.minimum(carry[j], rolled[G - 1 - j]) for j in range(G)]
        for a, b in _BITONIC16:
            lo = jnp.minimum(lows[a], lows[b])
            hi = jnp.maximum(lows[a], lows[b])
            lows[a], lows[b] = lo, hi
        carry = lows

    res = jnp.sqrt(
        jnp.concatenate([carry[j][:, 0:1] for j in range(G)], axis=1)
    )
    o_ref[t] = res


@jax.jit
def kernel(p):
    n = p.shape[0]
    assert n % B == 0
    x = p[:, 0]
    y = p[:, 1]
    span = G * LANES
    npad = (n + span - 1) // span * span
    s_rows = npad // LANES
    kx = jnp.full((npad,), PADV, jnp.float32).at[:n].set(x).reshape(s_rows, LANES)
    ky = jnp.full((npad,), PADV, jnp.float32).at[:n].set(y).reshape(s_rows, LANES)
    nb = n // B
    n_sub = 1
    for cand in (10, 5, 4, 2):
        if nb % cand == 0:
            n_sub = cand
            break
    qx = jnp.broadcast_to(x.reshape(nb, B)[:, :, None], (nb, B, LANES))
    qy = jnp.broadcast_to(y.reshape(nb, B)[:, :, None], (nb, B, LANES))

    kern = functools.partial(_knn_kernel, n_groups=s_rows // G, n_sub=n_sub)
    out = pl.pallas_call(
        kern,
        grid=(nb // n_sub,),
        in_specs=[
            pl.BlockSpec(kx.shape, lambda i: (0, 0)),
            pl.BlockSpec(ky.shape, lambda i: (0, 0)),
            pl.BlockSpec((n_sub, B, LANES), lambda i: (i, 0, 0)),
            pl.BlockSpec((n_sub, B, LANES), lambda i: (i, 0, 0)),
        ],
        out_specs=pl.BlockSpec((n_sub, B, 16), lambda i: (i, 0, 0)),
        out_shape=jax.ShapeDtypeStruct((nb, B, 16), jnp.float32),
    )(kx, ky, qx, qy)
    return out.reshape(n, 16)


# speedup vs baseline: 1.3972x; 1.3972x over previous
import functools

import jax
import jax.numpy as jnp
from jax.experimental import pallas as pl

BIG = 3.0e38
PADV = 1.0e18
B = 8
LANES = 128
G = 16
LG = 16
QB = LANES // LG
QSTEP = B * QB


def _batcher_pairs(n):
    pairs = []
    p = 1
    while p < n:
        k = p
        while k >= 1:
            for j in range(k % p, n - k, 2 * k):
                for i in range(0, min(k, n - j - k)):
                    if (i + j) // (2 * p) == (i + j + k) // (2 * p):
                        pairs.append((i + j, i + j + k))
            k //= 2
        p *= 2
    return pairs


_SORT16 = _batcher_pairs(16)
_BITONIC16 = [(j, j + d) for d in (8, 4, 2, 1) for j in range(16) if (j & d) == 0]


def _merge_lowest16(carry, other):
    lows = [jnp.minimum(carry[j], other[G - 1 - j]) for j in range(G)]
    for a, b in _BITONIC16:
        lo = jnp.minimum(lows[a], lows[b])
        hi = jnp.maximum(lows[a], lows[b])
        lows[a], lows[b] = lo, hi
    return lows


def _knn_kernel(kx_ref, ky_ref, qx_ref, qy_ref, o_ref, *, n_rows):
    i = pl.program_id(0)
    qx = qx_ref[0]
    qy = qy_ref[0]
    lane_iota = jax.lax.broadcasted_iota(jnp.int32, (1, LANES), 1)
    sub_iota = jax.lax.broadcasted_iota(jnp.int32, (B, 1), 0)
    lane_mod = lane_iota % LG
    qidx = i * QSTEP + (lane_iota // LG) * B + sub_iota

    def dist_row(r):
        kxr = kx_ref[r : r + 1, :]
        kyr = ky_ref[r : r + 1, :]
        dx = qx - kxr
        dy = qy - kyr
        d2 = dx * dx + dy * dy
        kidx = r * LG + lane_mod
        return jnp.where(kidx == qidx, BIG, d2)

    carry = [jnp.full((B, LANES), BIG, jnp.float32) for _ in range(G)]
    for g in range(n_rows // G):
        rows = [dist_row(g * G + t) for t in range(G)]
        for a, b in _SORT16:
            lo = jnp.minimum(rows[a], rows[b])
            hi = jnp.maximum(rows[a], rows[b])
            rows[a], rows[b] = lo, hi
        carry = _merge_lowest16(carry, rows)

    for d in (1, 2, 4, 8):
        rolled = [jnp.roll(c, -d, axis=1) for c in carry]
        carry = _merge_lowest16(carry, rolled)

    for j in range(QB):
        if j == 0:
            cols = [c[:, 0:1] for c in carry]
        else:
            cols = [jnp.roll(c, -(LG * j), axis=1)[:, 0:1] for c in carry]
        o_ref[0, j] = jnp.sqrt(jnp.concatenate(cols, axis=1))


@jax.jit
def kernel(p):
    n = p.shape[0]
    x = p[:, 0]
    y = p[:, 1]
    span = LG * G
    npad = (n + span - 1) // span * span
    n_rows = npad // LG
    kxp = jnp.full((npad,), PADV, jnp.float32).at[:n].set(x)
    kyp = jnp.full((npad,), PADV, jnp.float32).at[:n].set(y)
    kx = jnp.tile(kxp.reshape(n_rows, LG), (1, QB))
    ky = jnp.tile(kyp.reshape(n_rows, LG), (1, QB))
    nsteps = (n + QSTEP - 1) // QSTEP
    nq = nsteps * QSTEP
    qxp = jnp.zeros((nq,), jnp.float32).at[:n].set(x)
    qyp = jnp.zeros((nq,), jnp.float32).at[:n].set(y)

    def pack(q):
        q = q.reshape(nsteps, QB, B).transpose(0, 2, 1)
        return jnp.repeat(q, LG, axis=2)

    qx = pack(qxp)
    qy = pack(qyp)

    kern = functools.partial(_knn_kernel, n_rows=n_rows)
    out = pl.pallas_call(
        kern,
        grid=(nsteps,),
        in_specs=[
            pl.BlockSpec(kx.shape, lambda i: (0, 0)),
            pl.BlockSpec(ky.shape, lambda i: (0, 0)),
            pl.BlockSpec((1, B, LANES), lambda i: (i, 0, 0)),
            pl.BlockSpec((1, B, LANES), lambda i: (i, 0, 0)),
        ],
        out_specs=pl.BlockSpec((1, QB, B, 16), lambda i: (i, 0, 0, 0)),
        out_shape=jax.ShapeDtypeStruct((nsteps, QB, B, 16), jnp.float32),
    )(kx, ky, qx, qy)
    return out.reshape(nq, 16)[:n]
